# baseline (speedup 1.0000x reference)
import functools
import jax
import jax.numpy as jnp
from jax import lax
from jax.experimental import pallas as pl
from jax.experimental.pallas import tpu as pltpu
from jax.experimental.pallas import tpu_sc as plsc

VIS_LEN = 50
IMU_LEN = 200
SEQ = VIS_LEN + IMU_LEN
HIDDEN = 128
EPS = 1e-12
NC, NS, NW, L = 2, 16, 32, 16
CH = 256
NV = HIDDEN // L


def _srsqrt(v):
    i = lax.bitcast_convert_type(v, jnp.int32)
    i = jnp.int32(0x5F3759DF) - lax.shift_right_logical(i, 1)
    r = lax.bitcast_convert_type(i, jnp.float32)
    for _ in range(3):
        r = r * (1.5 - 0.5 * v * r * r)
    return r


def _token_ln(in_buf, out_buf, t, bias, gs, bs):
    xs = [in_buf[t, pl.ds(L * i, L)] + bias[i] for i in range(NV)]
    s1 = xs[0]
    s2 = xs[0] * xs[0]
    for i in range(1, NV):
        s1 = s1 + xs[i]
        s2 = s2 + xs[i] * xs[i]
    t1 = jnp.sum(s1)
    t2 = jnp.sum(s2)
    mu = t1 * (1.0 / HIDDEN)
    var = t2 * (1.0 / HIDDEN) - mu * mu
    r = _srsqrt(var + EPS)
    for i in range(NV):
        out_buf[t, pl.ds(L * i, L)] = (xs[i] - mu) * r * gs[i] + bs[i]


def _sc_body(vis_hbm, aud_hbm, pvt_hbm, pit_hbm, mt_hbm, e1_hbm, e2_hbm,
             g_hbm, b_hbm, out_hbm,
             in_buf, out_buf, pos_buf, mod_buf, g_buf, b_buf):
    B = 1024
    CPS = B // CH
    wid = lax.axis_index("s") * NC + lax.axis_index("c")
    pltpu.sync_copy(g_hbm, g_buf)
    pltpu.sync_copy(b_hbm, b_buf)
    gs = [g_buf[0, pl.ds(L * i, L)] for i in range(NV)]
    bs = [b_buf[0, pl.ds(L * i, L)] for i in range(NV)]

    def _chunk_ln():
        bias = [pos_buf[0, pl.ds(L * i, L)] + mod_buf[0, pl.ds(L * i, L)]
                for i in range(NV)]

        def body(t, c):
            _token_ln(in_buf, out_buf, t, bias, gs, bs)
            return c
        lax.fori_loop(0, CH, body, 0)

    def _seg(src_hbm, pos_hbm, mrow, out_s0, n_s):
        kmax = (n_s * CPS + NW - 1) // NW

        def body(k, c):
            g = wid + k * NW

            @pl.when(g < n_s * CPS)
            def _():
                s_rel = g // CPS
                b0 = (g % CPS) * CH
                pltpu.sync_copy(src_hbm.at[pl.ds(s_rel * B + b0, CH)], in_buf)
                pltpu.sync_copy(pos_hbm.at[pl.ds(1 + s_rel, 1)], pos_buf)
                pltpu.sync_copy(mt_hbm.at[pl.ds(mrow, 1)], mod_buf)
                _chunk_ln()
                pltpu.sync_copy(
                    out_buf, out_hbm.at[pl.ds((out_s0 + s_rel) * B + b0, CH)])
            return c
        lax.fori_loop(0, kmax, body, 0)

    def _esp(e_hbm, pos_hbm, mrow, out_s, cidx):
        pltpu.sync_copy(e_hbm, in_buf.at[pl.ds(0, 1)])
        pltpu.sync_copy(pos_hbm.at[pl.ds(0, 1)], pos_buf)
        pltpu.sync_copy(mt_hbm.at[pl.ds(mrow, 1)], mod_buf)
        bias = [pos_buf[0, pl.ds(L * i, L)] + mod_buf[0, pl.ds(L * i, L)]
                for i in range(NV)]
        _token_ln(in_buf, out_buf, 0, bias, gs, bs)
        ys = [out_buf[0, pl.ds(L * i, L)] for i in range(NV)]

        def fill(t, c):
            for i in range(NV):
                out_buf[t, pl.ds(L * i, L)] = ys[i]
            return c
        lax.fori_loop(1, CH, fill, 0)
        pltpu.sync_copy(out_buf, out_hbm.at[pl.ds(out_s * B + cidx * CH, CH)])

    @pl.when(wid < 4)
    def _():
        _esp(e1_hbm, pvt_hbm, 0, 0, wid)

    @pl.when((wid >= 4) & (wid < 8))
    def _():
        _esp(e2_hbm, pit_hbm, 1, VIS_LEN, wid - 4)

    _seg(vis_hbm, pvt_hbm, 0, 1, VIS_LEN - 1)
    _seg(aud_hbm, pit_hbm, 1, VIS_LEN + 1, IMU_LEN - 1)


def kernel(visual_embedding, audio_embedding, posi_visual_table,
           posi_imu_table, modal_table, esp_1, esp_2, ln_gamma, ln_beta):
    B = visual_embedding.shape[0]
    vis2 = jnp.transpose(visual_embedding, (1, 0, 2)).reshape(-1, HIDDEN)
    aud2 = jnp.transpose(audio_embedding, (1, 0, 2)).reshape(-1, HIDDEN)
    run = functools.partial(
        pl.kernel,
        mesh=plsc.VectorSubcoreMesh(core_axis_name="c", subcore_axis_name="s"),
        compiler_params=pltpu.CompilerParams(needs_layout_passes=False),
        out_type=jax.ShapeDtypeStruct((SEQ * B, HIDDEN), jnp.float32),
        scratch_types=[
            pltpu.VMEM((CH, HIDDEN), jnp.float32),
            pltpu.VMEM((CH, HIDDEN), jnp.float32),
            pltpu.VMEM((1, HIDDEN), jnp.float32),
            pltpu.VMEM((1, HIDDEN), jnp.float32),
            pltpu.VMEM((1, HIDDEN), jnp.float32),
            pltpu.VMEM((1, HIDDEN), jnp.float32),
        ],
    )(_sc_body)
    out2 = run(vis2, aud2, posi_visual_table, posi_imu_table, modal_table,
               esp_1.reshape(1, HIDDEN), esp_2.reshape(1, HIDDEN),
               ln_gamma.reshape(1, HIDDEN), ln_beta.reshape(1, HIDDEN))
    return jnp.transpose(out2.reshape(SEQ, B, HIDDEN), (1, 0, 2))


# speedup vs baseline: 2.2528x; 2.2528x over previous
import jax
import jax.numpy as jnp
from jax.experimental import pallas as pl
from jax.experimental.pallas import tpu as pltpu

VIS_LEN = 50
IMU_LEN = 200
SEQ = VIS_LEN + IMU_LEN
HIDDEN = 128
HSEQ = SEQ // 2
EPS = 1e-12
BBLK = 64
_INV_H = 1.0 / HIDDEN


def _ln_rows(x, g, b):
    mu = jnp.mean(x, axis=-1, keepdims=True)
    var = jnp.mean((x - mu) ** 2, axis=-1, keepdims=True)
    return (x - mu) * jax.lax.rsqrt(var + EPS) * g + b


def _ln_seg(v, c, g, b):
    cm = c[:, None, :]
    s1c = jnp.sum(c, axis=-1)[:, None, None]
    mu = (jnp.sum(v, axis=-1, keepdims=True) + s1c) * _INV_H
    xc = v + cm
    s2 = jnp.sum(xc * xc, axis=-1, keepdims=True) * _INV_H
    var = s2 - mu * mu
    r = jax.lax.rsqrt(var + EPS)
    return ((v - mu) + cm) * r * g + b


def _body(vis_ref, aud_ref, pvt_ref, pit_ref, mt_ref, e1_ref, e2_ref,
          g_ref, b_ref, out_ref):
    k = pl.program_id(1)
    bias_imu = pit_ref[...] + mt_ref[1:2, :]
    g = g_ref[...]
    b = b_ref[...]
    n = out_ref.shape[1]

    @pl.when(k == 0)
    def _first_half():
        bias_vis = pvt_ref[...] + mt_ref[0:1, :]
        y0 = _ln_rows(e1_ref[...] + bias_vis[0:1, :], g, b)
        out_ref[0:1] = jnp.broadcast_to(y0[:, None, :], (1, n, HIDDEN))
        out_ref[1:VIS_LEN] = _ln_seg(vis_ref[...], bias_vis[1:, :],
                                     g[None], b[None])
        y1 = _ln_rows(e2_ref[...] + bias_imu[0:1, :], g, b)
        out_ref[VIS_LEN:VIS_LEN + 1] = jnp.broadcast_to(
            y1[:, None, :], (1, n, HIDDEN))
        out_ref[VIS_LEN + 1:] = _ln_seg(aud_ref[0:HSEQ - VIS_LEN - 1],
                                        bias_imu[1:HSEQ - VIS_LEN, :],
                                        g[None], b[None])

    @pl.when(k == 1)
    def _second_half():
        out_ref[...] = _ln_seg(aud_ref[HSEQ - VIS_LEN - 1:],
                               bias_imu[HSEQ - VIS_LEN:, :],
                               g[None], b[None])


def kernel(visual_embedding, audio_embedding, posi_visual_table,
           posi_imu_table, modal_table, esp_1, esp_2, ln_gamma, ln_beta):
    B = visual_embedding.shape[0]
    vis_t = jnp.transpose(visual_embedding, (1, 0, 2))
    aud_t = jnp.transpose(audio_embedding, (1, 0, 2))
    grid = (B // BBLK, 2)
    out_t = pl.pallas_call(
        _body,
        grid=grid,
        in_specs=[
            pl.BlockSpec((VIS_LEN - 1, BBLK, HIDDEN), lambda j, k: (0, j, 0)),
            pl.BlockSpec((IMU_LEN - 1, BBLK, HIDDEN), lambda j, k: (0, j, 0)),
            pl.BlockSpec((VIS_LEN, HIDDEN), lambda j, k: (0, 0)),
            pl.BlockSpec((IMU_LEN, HIDDEN), lambda j, k: (0, 0)),
            pl.BlockSpec((2, HIDDEN), lambda j, k: (0, 0)),
            pl.BlockSpec((1, HIDDEN), lambda j, k: (0, 0)),
            pl.BlockSpec((1, HIDDEN), lambda j, k: (0, 0)),
            pl.BlockSpec((1, HIDDEN), lambda j, k: (0, 0)),
            pl.BlockSpec((1, HIDDEN), lambda j, k: (0, 0)),
        ],
        out_specs=pl.BlockSpec((HSEQ, BBLK, HIDDEN), lambda j, k: (k, j, 0)),
        out_shape=jax.ShapeDtypeStruct((SEQ, B, HIDDEN), jnp.float32),
        compiler_params=pltpu.CompilerParams(
            dimension_semantics=("parallel", "arbitrary"),
        ),
    )(
        vis_t,
        aud_t,
        posi_visual_table,
        posi_imu_table,
        modal_table,
        esp_1.reshape(1, HIDDEN),
        esp_2.reshape(1, HIDDEN),
        ln_gamma.reshape(1, HIDDEN),
        ln_beta.reshape(1, HIDDEN),
    )
    return jnp.transpose(out_t, (1, 0, 2))


# speedup vs baseline: 3.3238x; 1.4754x over previous
import jax
import jax.numpy as jnp
from jax.experimental import pallas as pl
from jax.experimental.pallas import tpu as pltpu

VIS_LEN = 50
IMU_LEN = 200
SEQ = VIS_LEN + IMU_LEN
HIDDEN = 128
EPS = 1e-12
BBLK = 64
_INV_H = 1.0 / HIDDEN


def _ln_rows(x, g, b):
    mu = jnp.mean(x, axis=-1, keepdims=True)
    var = jnp.mean((x - mu) ** 2, axis=-1, keepdims=True)
    return (x - mu) * jax.lax.rsqrt(var + EPS) * g + b


def _ln_seg(v, c, g, b):
    cm = c[:, None, :]
    s1c = jnp.sum(c, axis=-1)[:, None, None]
    mu = (jnp.sum(v, axis=-1, keepdims=True) + s1c) * _INV_H
    xc = v + cm
    s2 = jnp.sum(xc * xc, axis=-1, keepdims=True) * _INV_H
    var = s2 - mu * mu
    r = jax.lax.rsqrt(var + EPS)
    return ((v - mu) + cm) * r * g + b


def _body(vis_ref, aud_ref, pvt_ref, pit_ref, mt_ref, e1_ref, e2_ref,
          g_ref, b_ref, out_ref):
    bias_vis = pvt_ref[...] + mt_ref[0:1, :]
    bias_imu = pit_ref[...] + mt_ref[1:2, :]
    g = g_ref[...]
    b = b_ref[...]

    n = out_ref.shape[1]
    y0 = _ln_rows(e1_ref[...] + bias_vis[0:1, :], g, b)
    out_ref[0:1] = jnp.broadcast_to(y0[:, None, :], (1, n, HIDDEN))
    out_ref[1:VIS_LEN] = _ln_seg(vis_ref[...], bias_vis[1:, :], g[None], b[None])
    y1 = _ln_rows(e2_ref[...] + bias_imu[0:1, :], g, b)
    out_ref[VIS_LEN:VIS_LEN + 1] = jnp.broadcast_to(y1[:, None, :], (1, n, HIDDEN))
    out_ref[VIS_LEN + 1:] = _ln_seg(aud_ref[...], bias_imu[1:, :], g[None], b[None])


def kernel(visual_embedding, audio_embedding, posi_visual_table,
           posi_imu_table, modal_table, esp_1, esp_2, ln_gamma, ln_beta):
    B = visual_embedding.shape[0]
    vis_t = jnp.transpose(visual_embedding, (1, 0, 2))
    aud_t = jnp.transpose(audio_embedding, (1, 0, 2))
    grid = (B // BBLK,)
    out_t = pl.pallas_call(
        _body,
        grid=grid,
        in_specs=[
            pl.BlockSpec((VIS_LEN - 1, BBLK, HIDDEN), lambda j: (0, j, 0)),
            pl.BlockSpec((IMU_LEN - 1, BBLK, HIDDEN), lambda j: (0, j, 0)),
            pl.BlockSpec((VIS_LEN, HIDDEN), lambda j: (0, 0)),
            pl.BlockSpec((IMU_LEN, HIDDEN), lambda j: (0, 0)),
            pl.BlockSpec((2, HIDDEN), lambda j: (0, 0)),
            pl.BlockSpec((1, HIDDEN), lambda j: (0, 0)),
            pl.BlockSpec((1, HIDDEN), lambda j: (0, 0)),
            pl.BlockSpec((1, HIDDEN), lambda j: (0, 0)),
            pl.BlockSpec((1, HIDDEN), lambda j: (0, 0)),
        ],
        out_specs=pl.BlockSpec((SEQ, BBLK, HIDDEN), lambda j: (0, j, 0)),
        out_shape=jax.ShapeDtypeStruct((SEQ, B, HIDDEN), jnp.float32),
        compiler_params=pltpu.CompilerParams(
            dimension_semantics=("parallel",),
        ),
    )(
        vis_t,
        aud_t,
        posi_visual_table,
        posi_imu_table,
        modal_table,
        esp_1.reshape(1, HIDDEN),
        esp_2.reshape(1, HIDDEN),
        ln_gamma.reshape(1, HIDDEN),
        ln_beta.reshape(1, HIDDEN),
    )
    return jnp.transpose(out_t, (1, 0, 2))
